# baseline (speedup 1.0000x reference)
import functools

import jax
import jax.numpy as jnp
from jax import lax
from jax.experimental import pallas as pl
from jax.experimental.pallas import tpu as pltpu
from jax.experimental.pallas import tpu_sc as plsc

LANES = 16
TILE = 128
NCORES = 2
NSUB = 16
SC_CHUNK = 2


def _make_sc_pool(vocab, emdim):
    assert emdim % LANES == 0
    ej = emdim // LANES
    mesh = plsc.VectorSubcoreMesh(core_axis_name="c", subcore_axis_name="s")

    @functools.partial(
        pl.kernel,
        mesh=mesh,
        out_type=jax.ShapeDtypeStruct((NCORES * NSUB, emdim), jnp.float32),
        compiler_params=pltpu.CompilerParams(needs_layout_passes=False),
        scratch_types=[
            pltpu.VMEM((LANES,), jnp.int32),
            pltpu.VMEM((SC_CHUNK, emdim, TILE), jnp.float32),
            pltpu.VMEM((emdim,), jnp.float32),
            pltpu.SemaphoreType.DMA,
        ],
    )
    def pool(sent_hbm, tablet_hbm, out_hbm, idx_v, blocks_v, xv, sem):
        c = lax.axis_index("c")
        s = lax.axis_index("s")
        pltpu.sync_copy(sent_hbm.at[pl.ds((s // 2) * 8, 8)],
                        idx_v.at[pl.ds(0, 8)])
        vidx = idx_v[pl.ds(0, LANES)]
        ii = lax.iota(jnp.int32, LANES)

        t = 4 * (s % 2) + 2 * c
        rs = [jnp.where(t == 0, vidx[i],
              jnp.where(t == 2, vidx[i + 2],
              jnp.where(t == 4, vidx[i + 4], vidx[i + 6])))
              for i in range(SC_CHUNK)]
        for i in range(SC_CHUNK):
            off = pl.multiple_of((rs[i] // TILE) * TILE, TILE)
            pltpu.async_copy(tablet_hbm.at[:, pl.ds(off, TILE)],
                             blocks_v.at[i], sem)
        accs = [jnp.zeros((LANES,), jnp.float32) for _ in range(ej)]
        for i in range(SC_CHUNK):
            pltpu.make_async_copy(tablet_hbm.at[:, pl.ds(0, TILE)],
                                  blocks_v.at[i], sem).wait()
            col = jnp.full((LANES,), rs[i] % TILE, jnp.int32)
            for j in range(ej):
                accs[j] = accs[j] + plsc.load_gather(
                    blocks_v.at[i], [ii + LANES * j, col])
        for j in range(ej):
            xv[pl.ds(LANES * j, LANES)] = accs[j]
        pltpu.sync_copy(xv, out_hbm.at[c * NSUB + s])

    return pool


def _make_tc_gather(start, nwords, emdim):
    def body(idx_ref, tablet_ref, acc_ref, blocks, sem):
        for i in range(nwords):
            r = idx_ref[start + i]
            off = pl.multiple_of((r // TILE) * TILE, TILE)
            pltpu.async_copy(tablet_ref.at[:, pl.ds(off, TILE)],
                             blocks.at[i], sem)
        lane = jax.lax.broadcasted_iota(jnp.int32, (emdim, TILE), 1)
        acc = jnp.zeros((emdim, TILE), jnp.float32)
        for i in range(nwords):
            pltpu.make_async_copy(tablet_ref.at[:, pl.ds(0, TILE)],
                                  blocks.at[i], sem).wait()
            r = idx_ref[start + i]
            acc = acc + jnp.where(lane == r % TILE, blocks[i], 0.0)
        acc_ref[...] = jnp.sum(acc, axis=1).reshape(1, emdim)

    return pl.pallas_call(
        body,
        in_specs=[pl.BlockSpec(memory_space=pltpu.SMEM),
                  pl.BlockSpec(memory_space=pl.ANY)],
        out_shape=jax.ShapeDtypeStruct((1, emdim), jnp.float32),
        scratch_shapes=[pltpu.VMEM((nwords, emdim, TILE), jnp.float32),
                        pltpu.SemaphoreType.DMA],
    )


def _make_tc_finish(seq, emdim, hidden):
    def body(p_ref, acc_ref, w_ref, b_ref, o_ref):
        sc_part = jnp.sum(p_ref[...], axis=0).reshape(1, emdim)
        x = (sc_part + acc_ref[...]) * (1.0 / seq)
        y = lax.dot_general(x, w_ref[...], (((1,), (1,)), ((), ())),
                            preferred_element_type=jnp.float32)
        o_ref[...] = jnp.tanh(y + b_ref[...])

    return pl.pallas_call(
        body, out_shape=jax.ShapeDtypeStruct((1, hidden), jnp.float32))


def kernel(sentence, table, W, b):
    seq = sentence.shape[0]
    vocab, emdim = table.shape
    hidden = W.shape[0]
    sc_words = NCORES * NSUB * SC_CHUNK
    assert sc_words < seq
    tc_words = seq - sc_words
    sent = sentence.astype(jnp.int32)
    tablet = table.T
    psum = _make_sc_pool(vocab, emdim)(sent, tablet)
    acc = _make_tc_gather(sc_words, tc_words, emdim)(sent, tablet)
    return _make_tc_finish(seq, emdim, hidden)(psum, acc, W,
                                               b.reshape(1, hidden))
